# baseline (speedup 1.0000x reference)
import jax
import jax.numpy as jnp
from jax.experimental import pallas as pl
from jax.experimental.pallas import tpu as pltpu

_B, _K, _D, _E, _P = 128, 36, 2048, 1024, 5
_GB = 4
_RT = _GB * _K


def _body(imgs_ref, s_ref, rng_ref,
          gw1a_ref, gw1b_ref, gb1_ref, gw2_ref, gb2_ref,
          nw1a_ref, nw1b_ref, nb1_ref, nw2_ref, nb2_ref,
          mw1_ref, mb1_ref, mw2_ref, mb2_ref,
          out_ref):
    f32 = jnp.float32
    bf16 = jnp.bfloat16

    def dot_nn(a, b):
        return jax.lax.dot_general(a, b, (((1,), (0,)), ((), ())),
                                   preferred_element_type=f32)

    xbT = jnp.transpose(imgs_ref[...])
    xbT16 = xbT.astype(bf16)
    sbT16 = jnp.transpose(s_ref[...]).astype(bf16)

    hg = dot_nn(gw1a_ref[...], xbT16)
    hg = hg + dot_nn(gw1b_ref[...], sbT16)
    hg = jnp.maximum(hg + gb1_ref[...], 0.0)
    gate = jnp.sum(hg * gw2_ref[...], axis=0, keepdims=True) + gb2_ref[...]
    m = jax.nn.sigmoid(gate)

    hn = dot_nn(nw1a_ref[...], xbT16)
    hn = hn + dot_nn(nw1b_ref[...], sbT16)
    hn = jnp.maximum(hn + nb1_ref[...], 0.0).astype(bf16)
    v = dot_nn(nw2_ref[...], hn) + nb2_ref[...]
    mv = m * v

    r = rng_ref[...]
    jidx = jax.lax.broadcasted_iota(jnp.int32, (_K, _K), 0)
    cidx = jax.lax.broadcasted_iota(jnp.int32, (_K, _K), 1)
    tri = (jidx <= cidx).astype(f32)
    eye = (jidx == cidx).astype(f32)
    rank = dot_nn(r, tri)
    w = r * (rank <= float(_P)).astype(f32)
    deficit = jnp.maximum(float(_P) - rank[:, _K - 1:_K], 0.0)

    outs = []
    for b in range(_GB):
        wbT = jnp.transpose(w[b * _K:(b + 1) * _K, :]
                            + deficit[b * _K:(b + 1) * _K, :] * eye)
        mvbT = mv[:, b * _K:(b + 1) * _K]
        outs.append(dot_nn(mvbT, wbT))
    aggT = jnp.concatenate(outs, axis=1)

    norm = jnp.sqrt(jnp.sum(aggT * aggT, axis=0, keepdims=True)) + 1e-8
    images2 = xbT + aggT / norm

    h3 = dot_nn(mw1_ref[...], images2.astype(bf16))
    h3 = jnp.maximum(h3 + mb1_ref[...], 0.0).astype(bf16)
    emb = dot_nn(mw2_ref[...], h3) + mb2_ref[...]
    n2 = jnp.sqrt(jnp.sum(emb * emb, axis=0, keepdims=True)) + 1e-8
    out_ref[...] = jnp.transpose(emb / n2)


def kernel(images, bboxes, img_range, gw1, gb1, gw2, gb2,
           nw1, nb1, nw2, nb2, mw1, mb1, mw2, mb2):
    f32 = jnp.float32
    bf16 = jnp.bfloat16
    N = _B * _K

    area = (bboxes[:, :, 2] - bboxes[:, :, 0]) * (bboxes[:, :, 3] - bboxes[:, :, 1])
    s = jnp.concatenate([bboxes, area[:, :, None]], axis=2) * 0.1
    s = jnp.pad(s, ((0, 0), (0, 0), (0, 3))).reshape(N, 8)

    imgs = images.reshape(N, _D)
    rng2d = img_range.reshape(N, _K)

    gw1a = gw1[:, :_D].astype(bf16)
    gw1b = jnp.pad(gw1[:, _D:], ((0, 0), (0, 3))).astype(bf16)
    nw1a = nw1[:, :_D].astype(bf16)
    nw1b = jnp.pad(nw1[:, _D:], ((0, 0), (0, 3))).astype(bf16)
    gw2c = gw2.reshape(_D, 1)
    nw2c = nw2.astype(bf16)
    mw1c = mw1.astype(bf16)
    mw2c = mw2.astype(bf16)

    gb1r = gb1.reshape(_D, 1)
    gb2r = gb2.reshape(1, 1)
    nb1r = nb1.reshape(_D, 1)
    nb2r = nb2.reshape(_D, 1)
    mb1r = mb1.reshape(_D, 1)
    mb2r = mb2.reshape(_E, 1)

    row_spec = lambda cols: pl.BlockSpec((_RT, cols), lambda i: (i, 0))
    full_spec = lambda rows, cols: pl.BlockSpec((rows, cols), lambda i: (0, 0))

    out = pl.pallas_call(
        _body,
        grid=(_B // _GB,),
        in_specs=[
            row_spec(_D),
            row_spec(8),
            row_spec(_K),
            full_spec(_D, _D),
            full_spec(_D, 8),
            full_spec(_D, 1),
            full_spec(_D, 1),
            full_spec(1, 1),
            full_spec(_D, _D),
            full_spec(_D, 8),
            full_spec(_D, 1),
            full_spec(_D, _D),
            full_spec(_D, 1),
            full_spec(_D, _D),
            full_spec(_D, 1),
            full_spec(_E, _D),
            full_spec(_E, 1),
        ],
        out_specs=row_spec(_E),
        out_shape=jax.ShapeDtypeStruct((N, _E), f32),
        compiler_params=pltpu.CompilerParams(
            dimension_semantics=("arbitrary",),
        ),
    )(imgs, s, rng2d, gw1a, gw1b, gb1r, gw2c, gb2r,
      nw1a, nw1b, nb1r, nw2c, nb2r, mw1c, mb1r, mw2c, mb2r)

    return out.reshape(_B, _K, _E)


# speedup vs baseline: 1.6731x; 1.6731x over previous
import jax
import jax.numpy as jnp
from jax.experimental import pallas as pl
from jax.experimental.pallas import tpu as pltpu

_B, _K, _D, _E, _P = 128, 36, 2048, 1024, 5
_GB = 4
_RT = _GB * _K


def _body(imgs_ref, s_ref, rng_ref,
          gw1a_hbm, gw1b_ref, gb1_ref, gw2_ref, gb2_ref,
          nw1a_hbm, nw1b_ref, nb1_ref, nw2_hbm, nb2_ref,
          mw1_hbm, mb1_ref, mw2_hbm, mb2_ref,
          out_ref,
          stage, gw1a_t, nw1a_t, nw2_t, mw1_t, mw2_t, dma_sem):
    f32 = jnp.float32
    bf16 = jnp.bfloat16

    @pl.when(pl.program_id(0) == 0)
    def _load_weights():
        def load_t(hbm_ref, dst_ref, rows):
            st = stage.at[pl.ds(0, rows), :]
            cp = pltpu.make_async_copy(hbm_ref, st, dma_sem)
            cp.start()
            cp.wait()
            dst_ref[...] = jnp.transpose(st[...])
        load_t(gw1a_hbm, gw1a_t, _D)
        load_t(nw1a_hbm, nw1a_t, _D)
        load_t(nw2_hbm, nw2_t, _D)
        load_t(mw1_hbm, mw1_t, _D)
        load_t(mw2_hbm, mw2_t, _E)

    xb = imgs_ref[...]
    xb16 = xb.astype(bf16)
    sb16 = s_ref[...].astype(bf16)

    def dot_nn(a, b):
        return jax.lax.dot_general(a, b, (((1,), (0,)), ((), ())),
                                   preferred_element_type=f32)

    hg = dot_nn(xb16, gw1a_t[...])
    hg = hg + dot_nn(sb16, gw1b_ref[...])
    hg = jnp.maximum(hg + gb1_ref[...], 0.0)
    gate = jnp.sum(hg * gw2_ref[...], axis=1, keepdims=True) + gb2_ref[...]
    m = jax.nn.sigmoid(gate)

    hn = dot_nn(xb16, nw1a_t[...])
    hn = hn + dot_nn(sb16, nw1b_ref[...])
    hn = jnp.maximum(hn + nb1_ref[...], 0.0).astype(bf16)
    v = dot_nn(hn, nw2_t[...]) + nb2_ref[...]
    mv = m * v

    r = rng_ref[...]
    jidx = jax.lax.broadcasted_iota(jnp.int32, (_K, _K), 0)
    cidx = jax.lax.broadcasted_iota(jnp.int32, (_K, _K), 1)
    tri = (jidx <= cidx).astype(f32)
    eye = (jidx == cidx).astype(f32)
    rank = dot_nn(r, tri)
    w = r * (rank <= float(_P)).astype(f32)
    deficit = jnp.maximum(float(_P) - rank[:, _K - 1:_K], 0.0)

    outs = []
    for b in range(_GB):
        wb = w[b * _K:(b + 1) * _K, :] + deficit[b * _K:(b + 1) * _K, :] * eye
        mvb = mv[b * _K:(b + 1) * _K, :]
        outs.append(dot_nn(wb, mvb))
    agg = jnp.concatenate(outs, axis=0)

    norm = jnp.sqrt(jnp.sum(agg * agg, axis=1, keepdims=True)) + 1e-8
    images2 = xb + agg / norm

    h3 = dot_nn(images2.astype(bf16), mw1_t[...])
    h3 = jnp.maximum(h3 + mb1_ref[...], 0.0).astype(bf16)
    emb = dot_nn(h3, mw2_t[...]) + mb2_ref[...]
    n2 = jnp.sqrt(jnp.sum(emb * emb, axis=1, keepdims=True)) + 1e-8
    out_ref[...] = emb / n2


def kernel(images, bboxes, img_range, gw1, gb1, gw2, gb2,
           nw1, nb1, nw2, nb2, mw1, mb1, mw2, mb2):
    f32 = jnp.float32
    bf16 = jnp.bfloat16
    N = _B * _K

    area = (bboxes[:, :, 2] - bboxes[:, :, 0]) * (bboxes[:, :, 3] - bboxes[:, :, 1])
    s = jnp.concatenate([bboxes, area[:, :, None]], axis=2) * 0.1
    s = jnp.pad(s, ((0, 0), (0, 0), (0, 3))).reshape(N, 8)

    imgs = images.reshape(N, _D)
    rng2d = img_range.reshape(N, _K)

    gw1a = gw1[:, :_D].astype(bf16)
    gw1b = jnp.pad(gw1[:, _D:].T, ((0, 3), (0, 0))).astype(bf16)
    nw1a = nw1[:, :_D].astype(bf16)
    nw1b = jnp.pad(nw1[:, _D:].T, ((0, 3), (0, 0))).astype(bf16)
    nw2c = nw2.astype(bf16)
    mw1c = mw1.astype(bf16)
    mw2c = mw2.astype(bf16)

    gb1r = gb1.reshape(1, _D)
    gb2r = gb2.reshape(1, 1)
    nb1r = nb1.reshape(1, _D)
    nb2r = nb2.reshape(1, _D)
    mb1r = mb1.reshape(1, _D)
    mb2r = mb2.reshape(1, _E)

    row_spec = lambda cols: pl.BlockSpec((_RT, cols), lambda i: (i, 0))
    full_spec = lambda rows, cols: pl.BlockSpec((rows, cols), lambda i: (0, 0))
    hbm_spec = pl.BlockSpec(memory_space=pltpu.MemorySpace.HBM)

    out = pl.pallas_call(
        _body,
        grid=(_B // _GB,),
        in_specs=[
            row_spec(_D),
            row_spec(8),
            row_spec(_K),
            hbm_spec,
            full_spec(8, _D),
            full_spec(1, _D),
            full_spec(1, _D),
            full_spec(1, 1),
            hbm_spec,
            full_spec(8, _D),
            full_spec(1, _D),
            hbm_spec,
            full_spec(1, _D),
            hbm_spec,
            full_spec(1, _D),
            hbm_spec,
            full_spec(1, _E),
        ],
        out_specs=row_spec(_E),
        out_shape=jax.ShapeDtypeStruct((N, _E), f32),
        scratch_shapes=[
            pltpu.VMEM((_D, _D), bf16),
            pltpu.VMEM((_D, _D), bf16),
            pltpu.VMEM((_D, _D), bf16),
            pltpu.VMEM((_D, _D), bf16),
            pltpu.VMEM((_D, _D), bf16),
            pltpu.VMEM((_D, _E), bf16),
            pltpu.SemaphoreType.DMA,
        ],
        compiler_params=pltpu.CompilerParams(
            dimension_semantics=("arbitrary",),
        ),
    )(imgs, s, rng2d, gw1a, gw1b, gb1r, gw2, gb2r,
      nw1a, nw1b, nb1r, nw2c, nb2r, mw1c, mb1r, mw2c, mb2r)

    return out.reshape(_B, _K, _E)


# speedup vs baseline: 1.8169x; 1.0859x over previous
import jax
import jax.numpy as jnp
from jax.experimental import pallas as pl
from jax.experimental.pallas import tpu as pltpu

_B, _K, _D, _E, _P = 128, 36, 2048, 1024, 5
_GB = 4
_RT = _GB * _K


def _body(imgs_ref, s_ref, rng_ref,
          gw1a_ref, gw1b_ref, gb1_ref, gw2_ref, gb2_ref,
          nw1a_ref, nw1b_ref, nb1_ref, nw2_ref, nb2_ref,
          mw1_ref, mb1_ref, mw2_ref, mb2_ref,
          out_ref):
    f32 = jnp.float32
    bf16 = jnp.bfloat16

    xb = imgs_ref[...]
    xb16 = xb.astype(bf16)
    sb16 = s_ref[...].astype(bf16)

    hg = jnp.dot(xb16, gw1a_ref[...], preferred_element_type=f32)
    hg = hg + jnp.dot(sb16, gw1b_ref[...], preferred_element_type=f32)
    hg = jnp.maximum(hg + gb1_ref[...], 0.0)
    gate = jnp.sum(hg * gw2_ref[...], axis=1, keepdims=True) + gb2_ref[...]
    m = jax.nn.sigmoid(gate)

    hn = jnp.dot(xb16, nw1a_ref[...], preferred_element_type=f32)
    hn = hn + jnp.dot(sb16, nw1b_ref[...], preferred_element_type=f32)
    hn = jnp.maximum(hn + nb1_ref[...], 0.0).astype(bf16)
    v = jnp.dot(hn, nw2_ref[...], preferred_element_type=f32) + nb2_ref[...]
    mv = m * v

    r = rng_ref[...]
    jidx = jax.lax.broadcasted_iota(jnp.int32, (_K, _K), 0)
    cidx = jax.lax.broadcasted_iota(jnp.int32, (_K, _K), 1)
    tri = (jidx <= cidx).astype(f32)
    eye = (jidx == cidx).astype(f32)
    rank = jnp.dot(r, tri, preferred_element_type=f32)
    w = r * (rank <= float(_P)).astype(f32)
    deficit = jnp.maximum(float(_P) - rank[:, _K - 1:_K], 0.0)

    outs = []
    for b in range(_GB):
        wb = w[b * _K:(b + 1) * _K, :] + deficit[b * _K:(b + 1) * _K, :] * eye
        mvb = mv[b * _K:(b + 1) * _K, :]
        outs.append(jnp.dot(wb, mvb, preferred_element_type=f32))
    agg = jnp.concatenate(outs, axis=0)

    norm = jnp.sqrt(jnp.sum(agg * agg, axis=1, keepdims=True)) + 1e-8
    images2 = xb + agg / norm

    h3 = jnp.dot(images2.astype(bf16), mw1_ref[...], preferred_element_type=f32)
    h3 = jnp.maximum(h3 + mb1_ref[...], 0.0).astype(bf16)
    emb = jnp.dot(h3, mw2_ref[...], preferred_element_type=f32) + mb2_ref[...]
    n2 = jnp.sqrt(jnp.sum(emb * emb, axis=1, keepdims=True)) + 1e-8
    out_ref[...] = emb / n2


def kernel(images, bboxes, img_range, gw1, gb1, gw2, gb2,
           nw1, nb1, nw2, nb2, mw1, mb1, mw2, mb2):
    f32 = jnp.float32
    bf16 = jnp.bfloat16
    N = _B * _K

    area = (bboxes[:, :, 2] - bboxes[:, :, 0]) * (bboxes[:, :, 3] - bboxes[:, :, 1])
    s = jnp.concatenate([bboxes, area[:, :, None]], axis=2) * 0.1
    s = jnp.pad(s, ((0, 0), (0, 0), (0, 3))).reshape(N, 8)

    imgs = images.reshape(N, _D)
    rng2d = img_range.reshape(N, _K)

    gw1a = gw1[:, :_D].T.astype(bf16)
    gw1b = jnp.pad(gw1[:, _D:].T, ((0, 3), (0, 0))).astype(bf16)
    nw1a = nw1[:, :_D].T.astype(bf16)
    nw1b = jnp.pad(nw1[:, _D:].T, ((0, 3), (0, 0))).astype(bf16)
    nw2t = nw2.T.astype(bf16)
    mw1t = mw1.T.astype(bf16)
    mw2t = mw2.T.astype(bf16)

    gb1r = gb1.reshape(1, _D)
    gb2r = gb2.reshape(1, 1)
    nb1r = nb1.reshape(1, _D)
    nb2r = nb2.reshape(1, _D)
    mb1r = mb1.reshape(1, _D)
    mb2r = mb2.reshape(1, _E)

    row_spec = lambda cols: pl.BlockSpec((_RT, cols), lambda i: (i, 0))
    full_spec = lambda rows, cols: pl.BlockSpec((rows, cols), lambda i: (0, 0))

    out = pl.pallas_call(
        _body,
        grid=(_B // _GB,),
        in_specs=[
            row_spec(_D),
            row_spec(8),
            row_spec(_K),
            full_spec(_D, _D),
            full_spec(8, _D),
            full_spec(1, _D),
            full_spec(1, _D),
            full_spec(1, 1),
            full_spec(_D, _D),
            full_spec(8, _D),
            full_spec(1, _D),
            full_spec(_D, _D),
            full_spec(1, _D),
            full_spec(_D, _D),
            full_spec(1, _D),
            full_spec(_D, _E),
            full_spec(1, _E),
        ],
        out_specs=row_spec(_E),
        out_shape=jax.ShapeDtypeStruct((N, _E), f32),
        compiler_params=pltpu.CompilerParams(
            dimension_semantics=("arbitrary",),
        ),
    )(imgs, s, rng2d, gw1a, gw1b, gb1r, gw2, gb2r,
      nw1a, nw1b, nb1r, nw2t, nb2r, mw1t, mb1r, mw2t, mb2r)

    return out.reshape(_B, _K, _E)
